# baseline (speedup 1.0000x reference)
import functools

import jax
import jax.numpy as jnp
from jax import lax
from jax.experimental import pallas as pl
from jax.experimental.pallas import tpu as pltpu
from jax.experimental.pallas import tpu_sc as plsc

NC = 2
NS = 16
NW = NC * NS
LANES = 16
G = 64
NG = 32
D = 128
NACC = 8


def _make_sc_kernel(n_rows, n_groups, inv_count):
    mesh = plsc.VectorSubcoreMesh(core_axis_name="c", subcore_axis_name="s")
    scale = jnp.float32(0.1 * inv_count)

    @functools.partial(
        pl.kernel,
        out_type=jax.ShapeDtypeStruct((NW, LANES), jnp.float32),
        mesh=mesh,
        compiler_params=pltpu.CompilerParams(use_tc_tiling_on_sc=False),
        scratch_types=[
            pltpu.VMEM((NG, 2, G), jnp.int32),
            pltpu.VMEM((G, D // 2), jnp.int32),
            pltpu.VMEM((G, D // 2), jnp.int32),
            pltpu.VMEM((G, D // 2), jnp.int32),
            pltpu.VMEM((G, D // 2), jnp.int32),
            pltpu.VMEM((LANES,), jnp.float32),
            pltpu.VMEM_SHARED((n_rows, D // 2), jnp.int32),
            pltpu.SemaphoreType.DMA,
            pltpu.SemaphoreType.DMA,
        ],
    )
    def k(emb_hbm, idx_hbm, out_hbm,
          idxg, srows0, drows0, srows1, drows1,
          accv, emb_sp, sem0, sem1):
        wid = lax.axis_index("s") * NC + lax.axis_index("c")
        sid = lax.axis_index("s")
        srows = (srows0, srows1)
        drows = (drows0, drows1)
        sems = (sem0, sem1)

        @pl.when(sid == 0)
        def _():
            pltpu.sync_copy(emb_hbm, emb_sp)

        plsc.subcore_barrier()

        def start(chunk, b):
            pltpu.async_copy(emb_sp.at[idxg.at[chunk, 0]], srows[b], sems[b])
            pltpu.async_copy(emb_sp.at[idxg.at[chunk, 1]], drows[b], sems[b])

        def wait(chunk, b):
            pltpu.make_async_copy(emb_sp.at[idxg.at[chunk, 0]], srows[b],
                                  sems[b]).wait()
            pltpu.make_async_copy(emb_sp.at[idxg.at[chunk, 1]], drows[b],
                                  sems[b]).wait()

        def reduce_chunk(b, accs):
            sr = srows[b]
            dr = drows[b]

            @plsc.parallel_loop(0, G, carry=accs)
            def accs_out(i, a):
                new = list(a)
                mask = jnp.int32(-65536)
                for j in range(D // 32):
                    s = sr[i, pl.ds(j * 16, 16)]
                    t = dr[i, pl.ds(j * 16, 16)]
                    shi = lax.bitcast_convert_type(s & mask, jnp.float32)
                    thi = lax.bitcast_convert_type(t & mask, jnp.float32)
                    slo = lax.bitcast_convert_type(s << 16, jnp.float32)
                    tlo = lax.bitcast_convert_type(t << 16, jnp.float32)
                    fh = shi - thi
                    fl = slo - tlo
                    new[2 * j] = new[2 * j] + fh * fh
                    new[2 * j + 1] = new[2 * j + 1] + fl * fl
                return tuple(new)

            return accs_out

        n_pairs = NG // 2
        accs = tuple(jnp.zeros((LANES,), jnp.float32) for _ in range(NACC))

        def pair_body(t, accs):
            wait(2 * t, 0)
            accs = reduce_chunk(0, accs)

            @pl.when(t + 1 < n_pairs)
            def _():
                start(2 * t + 2, 0)

            wait(2 * t + 1, 1)
            accs = reduce_chunk(1, accs)

            @pl.when(t + 1 < n_pairs)
            def _():
                start(2 * t + 3, 1)

            return accs

        for g in range(n_groups):
            pltpu.sync_copy(idx_hbm.at[wid, g], idxg)
            start(0, 0)
            start(1, 1)
            accs = lax.fori_loop(0, n_pairs, pair_body, accs)

        acc = accs[0]
        for j in range(1, NACC):
            acc = acc + accs[j]
        accv[...] = acc * scale
        pltpu.sync_copy(accv, out_hbm.at[wid])

    return k


@jax.jit
def kernel(embeddings, edge_index):
    n_edges = edge_index.shape[1]
    span = NW * G * NG
    n_pad = ((n_edges + span - 1) // span) * span
    n_groups = n_pad // span

    ei = edge_index.astype(jnp.int32)
    pad = n_pad - n_edges
    src = jnp.pad(ei[0], (0, pad))
    dst = jnp.pad(ei[1], (0, pad))
    idx = jnp.stack([src.reshape(NW, n_groups, NG, G),
                     dst.reshape(NW, n_groups, NG, G)], axis=3)

    inv_count = 1.0 / (n_edges * embeddings.shape[1])
    k = _make_sc_kernel(embeddings.shape[0], n_groups, inv_count)
    emb_bf = embeddings.astype(jnp.bfloat16)
    emb_packed = jax.lax.bitcast_convert_type(
        emb_bf.reshape(embeddings.shape[0], D // 2, 2), jnp.int32)
    partials = k(emb_packed, idx)
    return jnp.sum(partials)
